# baseline (speedup 1.0000x reference)
import functools

import jax
import jax.numpy as jnp
import numpy as np
from jax import lax
from jax.experimental import pallas as pl
from jax.experimental.pallas import tpu as pltpu
from jax.experimental.pallas import tpu_sc as plsc

B, C, N = 2, 1024, 4096
GROUP_SIZES = (128, 128, 256, 512)
GROUP_STARTS = (0, 128, 256, 512)

NC, NS = 2, 16
NW = NC * NS
ROWS_PER_LEAF_PER_W = tuple(2 * g // NW for g in GROUP_SIZES)
IDX_PER_W = 2 * C // NW
CHUNK = 8
NBUF = 3

_RM = np.zeros(2 * C, dtype=np.int32)
_OM = np.zeros(2 * C, dtype=np.int32)
for _w in range(NW):
    _l = 0
    for _g, _s0, _rpw in zip(GROUP_SIZES, GROUP_STARTS, ROWS_PER_LEAF_PER_W):
        for _r in range(_rpw):
            _flat = _w * _rpw + _r
            _b, _j = divmod(_flat, _g)
            _RM[_w * IDX_PER_W + _l] = _s0 + _j
            _OM[_w * IDX_PER_W + _l] = _b * C
            _l += 1


def _rank_body(e_ref, rm_ref, om_ref, idx_ref, gidx_ref):
    e = e_ref[...]
    e_lanes = jnp.broadcast_to(e, (C, C))
    e_rows = lax.broadcast_in_dim(e.reshape(C), (C, C), (0,))
    ii = lax.broadcasted_iota(jnp.int32, (C, C), 0)
    jj = lax.broadcasted_iota(jnp.int32, (C, C), 1)
    before = (e_rows > e_lanes) | ((e_rows == e_lanes) & (ii < jj))
    rank = jnp.sum(before.astype(jnp.int32), axis=0)
    rank_lanes = rank.reshape(1, C)

    rr = lax.broadcasted_iota(jnp.int32, (C, C), 0)
    onehot = (jnp.broadcast_to(rank_lanes, (C, C)) == rr).astype(jnp.int32)
    sorted_idx = jnp.sum(onehot * jj, axis=1)
    idx_ref[...] = sorted_idx.reshape(C, 1)

    rm = rm_ref[...]
    om = om_ref[...]
    jj2 = lax.broadcasted_iota(jnp.int32, (2 * C, C), 1)
    oh2 = (jnp.broadcast_to(rank_lanes, (2 * C, C)) == rm).astype(jnp.int32)
    gidx_ref[...] = (jnp.sum(oh2 * jj2, axis=1).reshape(2 * C, 1) + om)


def _sort_and_index(e):
    return pl.pallas_call(
        _rank_body,
        out_shape=(jax.ShapeDtypeStruct((C, 1), jnp.int32),
                   jax.ShapeDtypeStruct((2 * C, 1), jnp.int32)),
    )(e.reshape(1, C), jnp.asarray(_RM).reshape(2 * C, 1),
      jnp.asarray(_OM).reshape(2 * C, 1))


def _gather_body(q_hbm, k_hbm, v_hbm, gidx_hbm, *refs):
    outs = (refs[0:4], refs[4:8], refs[8:12])
    idx_v = refs[12]
    bufs = refs[13:13 + NBUF]
    gsems = refs[13 + NBUF:13 + 2 * NBUF]
    wsems = refs[13 + 2 * NBUF:13 + 3 * NBUF]
    tables = (q_hbm, k_hbm, v_hbm)

    wid = lax.axis_index("s") * NC + lax.axis_index("c")
    pltpu.sync_copy(gidx_hbm.at[pl.ds(wid * IDX_PER_W, IDX_PER_W)], idx_v)

    sched = []
    for t in range(3):
        off = 0
        for g in range(4):
            rpw = ROWS_PER_LEAF_PER_W[g]
            for c in range(rpw // CHUNK):
                sched.append((t, g, off + c * CHUNK, c))
            off += rpw
    S = len(sched)

    def gather(s):
        t, g, ioff, c = sched[s]
        return pltpu.async_copy(
            tables[t].at[idx_v.at[pl.ds(ioff, CHUNK)]],
            bufs[s % NBUF], gsems[s % NBUF])

    def write(s):
        t, g, ioff, c = sched[s]
        row0 = wid * ROWS_PER_LEAF_PER_W[g] + c * CHUNK
        return pltpu.async_copy(
            bufs[s % NBUF], outs[t][g].at[pl.ds(row0, CHUNK)],
            wsems[s % NBUF])

    gh = [None] * S
    wh = [None] * S
    for s in range(min(2, S)):
        gh[s] = gather(s)
    for s in range(S):
        gh[s].wait()
        wh[s] = write(s)
        if s - 1 >= 0:
            wh[s - 1].wait()
        if s + 2 < S:
            gh[s + 2] = gather(s + 2)
    wh[S - 1].wait()


def kernel(q, k, v):
    energy = jnp.mean(jnp.mean(jnp.abs(q), axis=-1), axis=0)
    sorted_idx, gidx = _sort_and_index(energy)
    sorted_idx = sorted_idx.reshape(C)
    gidx = gidx.reshape(2 * C)

    idx_groups = [lax.slice(sorted_idx, (s0,), (s0 + g,))
                  for g, s0 in zip(GROUP_SIZES, GROUP_STARTS)]

    out_type = [jax.ShapeDtypeStruct((2 * g, N), jnp.float32)
                for g in GROUP_SIZES] * 3

    gather_call = functools.partial(
        pl.kernel,
        mesh=plsc.VectorSubcoreMesh(core_axis_name="c", subcore_axis_name="s"),
        out_type=out_type,
        scratch_types=(
            [pltpu.VMEM((IDX_PER_W,), jnp.int32)]
            + [pltpu.VMEM((CHUNK, N), jnp.float32)] * NBUF
            + [pltpu.SemaphoreType.DMA] * (2 * NBUF)
        ),
    )(_gather_body)

    flat = gather_call(q.reshape(B * C, N), k.reshape(B * C, N),
                       v.reshape(B * C, N), gidx)
    q_groups = [flat[i].reshape(B, g, N) for i, g in enumerate(GROUP_SIZES)]
    k_groups = [flat[4 + i].reshape(B, g, N) for i, g in enumerate(GROUP_SIZES)]
    v_groups = [flat[8 + i].reshape(B, g, N) for i, g in enumerate(GROUP_SIZES)]
    return tuple(q_groups) + tuple(k_groups) + tuple(v_groups) + tuple(idx_groups)
